# baseline (speedup 1.0000x reference)
import functools

import jax
import jax.numpy as jnp
from jax import lax
from jax.experimental import pallas as pl
from jax.experimental.pallas import tpu as pltpu
from jax.experimental.pallas import tpu_sc as plsc


def _sc_gather(table, idx2d):
    nr, il = idx2d.shape
    v, d = table.shape
    info = plsc.get_sparse_core_info()
    nw = info.num_cores * info.num_subcores
    rows_per_w = nr // nw

    mesh = plsc.VectorSubcoreMesh(core_axis_name="c", subcore_axis_name="s")

    @functools.partial(
        pl.kernel,
        mesh=mesh,
        out_type=jax.ShapeDtypeStruct((nr * il, d), table.dtype),
        scratch_types=[
            pltpu.VMEM((rows_per_w, il), jnp.int32),
            pltpu.VMEM((rows_per_w * il, d), table.dtype),
            pltpu.SemaphoreType.DMA,
        ],
    )
    def k(table_hbm, idx_hbm, out_hbm, idx_v, rows_v, sem):
        wid = lax.axis_index("s") * info.num_cores + lax.axis_index("c")
        base = wid * rows_per_w
        pltpu.sync_copy(idx_hbm.at[pl.ds(base, rows_per_w)], idx_v)
        copies = []
        for j in range(rows_per_w):
            copies.append(
                pltpu.async_copy(
                    table_hbm.at[idx_v.at[j]], rows_v.at[pl.ds(j * il, il)], sem
                )
            )
        for c in copies:
            c.wait()
        pltpu.sync_copy(rows_v, out_hbm.at[pl.ds(base * il, rows_per_w * il)])

    return k(table, idx2d)


def _mlp(e, W1, b1, W2, b2, block_m, block_n):
    b, k = e.shape
    h = W1.shape[1]
    v = W2.shape[1]
    nb = b // block_m
    nv = pl.cdiv(v, block_n)

    def body(e_ref, w1_ref, b1_ref, w2_ref, b2_ref, out_ref, h_ref):
        @pl.when(pl.program_id(1) == 0)
        def _():
            h_ref[...] = jnp.tanh(
                jnp.dot(e_ref[...], w1_ref[...], preferred_element_type=jnp.float32)
                + b1_ref[...]
            )

        out_ref[...] = (
            jnp.dot(h_ref[...], w2_ref[...], preferred_element_type=jnp.float32)
            + b2_ref[...]
        )

    return pl.pallas_call(
        body,
        grid=(nb, nv),
        in_specs=[
            pl.BlockSpec((block_m, k), lambda i, j: (i, 0)),
            pl.BlockSpec((k, h), lambda i, j: (0, 0)),
            pl.BlockSpec((1, h), lambda i, j: (0, 0)),
            pl.BlockSpec((h, block_n), lambda i, j: (0, j)),
            pl.BlockSpec((1, block_n), lambda i, j: (0, j)),
        ],
        out_specs=pl.BlockSpec((block_m, block_n), lambda i, j: (i, j)),
        out_shape=jax.ShapeDtypeStruct((b, v), jnp.float32),
        scratch_shapes=[pltpu.VMEM((block_m, h), jnp.float32)],
    )(e, W1, b1, W2, b2)


def kernel(x, embed, W1, b1, W2, b2):
    b, w = x.shape
    v, d = embed.shape
    h = W1.shape[1]
    idx = x.reshape(-1).astype(jnp.int32).reshape(-1, 128)
    rows = _sc_gather(embed, idx)
    e = rows.reshape(b, w * d)
    return _mlp(
        e,
        W1,
        b1.reshape(1, h),
        W2,
        b2.reshape(1, v),
        block_m=2048,
        block_n=2048,
    )


# speedup vs baseline: 1.0179x; 1.0179x over previous
import functools

import jax
import jax.numpy as jnp
from jax import lax
from jax.experimental import pallas as pl
from jax.experimental.pallas import tpu as pltpu
from jax.experimental.pallas import tpu_sc as plsc


def _sc_gather(table, idx2d):
    nr, il = idx2d.shape
    v, d = table.shape
    info = plsc.get_sparse_core_info()
    nw = info.num_cores * info.num_subcores
    rows_per_w = nr // nw

    mesh = plsc.VectorSubcoreMesh(core_axis_name="c", subcore_axis_name="s")

    @functools.partial(
        pl.kernel,
        mesh=mesh,
        out_type=jax.ShapeDtypeStruct((nr * il, d), table.dtype),
        scratch_types=[
            pltpu.VMEM((rows_per_w, il), jnp.int32),
            pltpu.VMEM((rows_per_w * il, d), table.dtype),
            pltpu.SemaphoreType.DMA,
        ],
    )
    def k(table_hbm, idx_hbm, out_hbm, idx_v, rows_v, sem):
        wid = lax.axis_index("s") * info.num_cores + lax.axis_index("c")
        base = wid * rows_per_w
        pltpu.sync_copy(idx_hbm.at[pl.ds(base, rows_per_w)], idx_v)
        copies = []
        for j in range(rows_per_w):
            copies.append(
                pltpu.async_copy(
                    table_hbm.at[idx_v.at[j]], rows_v.at[pl.ds(j * il, il)], sem
                )
            )
        for c in copies:
            c.wait()
        pltpu.sync_copy(rows_v, out_hbm.at[pl.ds(base * il, rows_per_w * il)])

    return k(table, idx2d)


def _mlp(e, W1, b1, W2, b2, block_m, block_n):
    b, k = e.shape
    h = W1.shape[1]
    v = W2.shape[1]
    nb = b // block_m
    nv = pl.cdiv(v, block_n)

    def body(e_ref, w1_ref, b1_ref, w2_ref, b2_ref, out_ref, h_ref):
        @pl.when(pl.program_id(1) == 0)
        def _():
            h_ref[...] = jnp.tanh(
                jnp.dot(e_ref[...], w1_ref[...], preferred_element_type=jnp.float32)
                + b1_ref[...]
            )

        out_ref[...] = (
            jnp.dot(h_ref[...], w2_ref[...], preferred_element_type=jnp.float32)
            + b2_ref[...]
        )

    return pl.pallas_call(
        body,
        grid=(nb, nv),
        in_specs=[
            pl.BlockSpec((block_m, k), lambda i, j: (i, 0)),
            pl.BlockSpec((k, h), lambda i, j: (0, 0)),
            pl.BlockSpec((1, h), lambda i, j: (0, 0)),
            pl.BlockSpec((h, block_n), lambda i, j: (0, j)),
            pl.BlockSpec((1, block_n), lambda i, j: (0, j)),
        ],
        out_specs=pl.BlockSpec((block_m, block_n), lambda i, j: (i, j)),
        out_shape=jax.ShapeDtypeStruct((b, v), jnp.float32),
        scratch_shapes=[pltpu.VMEM((block_m, h), jnp.float32)],
    )(e, W1, b1, W2, b2)


def kernel(x, embed, W1, b1, W2, b2):
    b, w = x.shape
    v, d = embed.shape
    h = W1.shape[1]
    idx = x.reshape(-1).astype(jnp.int32).reshape(-1, 128)
    rows = _sc_gather(embed, idx)
    e = rows.reshape(b, w * d)
    return _mlp(
        e,
        W1,
        b1.reshape(1, h),
        W2,
        b2.reshape(1, v),
        block_m=b,
        block_n=1536,
    )


# speedup vs baseline: 1.0236x; 1.0056x over previous
import functools

import jax
import jax.numpy as jnp
from jax import lax
from jax.experimental import pallas as pl
from jax.experimental.pallas import tpu as pltpu
from jax.experimental.pallas import tpu_sc as plsc


def _sc_gather(table, idx2d):
    nr, il = idx2d.shape
    v, d = table.shape
    info = plsc.get_sparse_core_info()
    nw = info.num_cores * info.num_subcores
    rows_per_w = nr // nw

    mesh = plsc.VectorSubcoreMesh(core_axis_name="c", subcore_axis_name="s")

    @functools.partial(
        pl.kernel,
        mesh=mesh,
        out_type=jax.ShapeDtypeStruct((nr * il, d), table.dtype),
        scratch_types=[
            pltpu.VMEM((rows_per_w, il), jnp.int32),
            pltpu.VMEM((rows_per_w * il, d), table.dtype),
            pltpu.SemaphoreType.DMA,
        ],
    )
    def k(table_hbm, idx_hbm, out_hbm, idx_v, rows_v, sem):
        wid = lax.axis_index("s") * info.num_cores + lax.axis_index("c")
        base = wid * rows_per_w
        pltpu.sync_copy(idx_hbm.at[pl.ds(base, rows_per_w)], idx_v)
        copies = []
        for j in range(rows_per_w):
            copies.append(
                pltpu.async_copy(
                    table_hbm.at[idx_v.at[j]], rows_v.at[pl.ds(j * il, il)], sem
                )
            )
        for c in copies:
            c.wait()
        pltpu.sync_copy(rows_v, out_hbm.at[pl.ds(base * il, rows_per_w * il)])

    return k(table, idx2d)


def _mlp(e, W1, b1, W2, b2, block_m, block_n):
    b, k = e.shape
    h = W1.shape[1]
    v = W2.shape[1]
    nb = b // block_m
    nv = pl.cdiv(v, block_n)

    def body(e_ref, w1_ref, b1_ref, w2_ref, b2_ref, out_ref, h_ref):
        @pl.when(pl.program_id(1) == 0)
        def _():
            h_ref[...] = jnp.tanh(
                jnp.dot(e_ref[...], w1_ref[...], preferred_element_type=jnp.float32)
                + b1_ref[...]
            )

        out_ref[...] = (
            jnp.dot(h_ref[...], w2_ref[...], preferred_element_type=jnp.float32)
            + b2_ref[...]
        )

    return pl.pallas_call(
        body,
        grid=(nb, nv),
        in_specs=[
            pl.BlockSpec((block_m, k), lambda i, j: (i, 0)),
            pl.BlockSpec((k, h), lambda i, j: (0, 0)),
            pl.BlockSpec((1, h), lambda i, j: (0, 0)),
            pl.BlockSpec((h, block_n), lambda i, j: (0, j)),
            pl.BlockSpec((1, block_n), lambda i, j: (0, j)),
        ],
        out_specs=pl.BlockSpec((block_m, block_n), lambda i, j: (i, j)),
        out_shape=jax.ShapeDtypeStruct((b, v), jnp.float32),
        scratch_shapes=[pltpu.VMEM((block_m, h), jnp.float32)],
    )(e, W1, b1, W2, b2)


def _mlp_manual(e, W1, b1, W2, b2, block_n):
    b, k = e.shape
    h = W1.shape[1]
    v = W2.shape[1]
    nv = pl.cdiv(v, block_n)
    n_full = v // block_n
    rem = v - n_full * block_n
    rem_al = (rem // 128) * 128
    rem_tl = rem - rem_al

    def body(e_ref, w1_ref, b1_ref, w2_ref, b2_ref, out_hbm, h_ref, obuf, tailbuf, sem):
        j = pl.program_id(0)
        slot = jax.lax.rem(j, 2)

        @pl.when(j == 0)
        def _():
            h_ref[...] = jnp.tanh(
                jnp.dot(e_ref[...], w1_ref[...], preferred_element_type=jnp.float32)
                + b1_ref[...]
            )

        @pl.when(j >= 2)
        def _():
            pltpu.make_async_copy(
                obuf.at[slot],
                out_hbm.at[:, pl.ds((j - 2) * block_n, block_n)],
                sem.at[slot],
            ).wait()

        obuf[slot] = (
            jnp.dot(h_ref[...], w2_ref[...], preferred_element_type=jnp.float32)
            + b2_ref[...]
        )

        @pl.when(j < n_full)
        def _():
            pltpu.make_async_copy(
                obuf.at[slot],
                out_hbm.at[:, pl.ds(j * block_n, block_n)],
                sem.at[slot],
            ).start()

        @pl.when(j == nv - 1)
        def _():
            if rem > 0:
                if rem_al > 0:
                    pltpu.make_async_copy(
                        obuf.at[slot, :, pl.ds(0, rem_al)],
                        out_hbm.at[:, pl.ds(n_full * block_n, rem_al)],
                        sem.at[slot],
                    ).start()
                    pltpu.make_async_copy(
                        obuf.at[slot, :, pl.ds(0, rem_al)],
                        out_hbm.at[:, pl.ds(n_full * block_n, rem_al)],
                        sem.at[slot],
                    ).wait()
                if rem_tl > 0:
                    tailbuf[...] = obuf[slot, :, rem_al : rem_al + rem_tl]
                    pltpu.make_async_copy(
                        tailbuf,
                        out_hbm.at[:, pl.ds(n_full * block_n + rem_al, rem_tl)],
                        sem.at[slot],
                    ).start()
                    pltpu.make_async_copy(
                        tailbuf,
                        out_hbm.at[:, pl.ds(n_full * block_n + rem_al, rem_tl)],
                        sem.at[slot],
                    ).wait()
            pltpu.make_async_copy(
                obuf.at[1 - slot],
                out_hbm.at[:, pl.ds((nv - 2) * block_n, block_n)],
                sem.at[1 - slot],
            ).wait()

    return pl.pallas_call(
        body,
        grid=(nv,),
        in_specs=[
            pl.BlockSpec((b, k), lambda j: (0, 0)),
            pl.BlockSpec((k, h), lambda j: (0, 0)),
            pl.BlockSpec((1, h), lambda j: (0, 0)),
            pl.BlockSpec((h, block_n), lambda j: (0, j)),
            pl.BlockSpec((1, block_n), lambda j: (0, j)),
        ],
        out_specs=pl.BlockSpec(memory_space=pl.ANY),
        out_shape=jax.ShapeDtypeStruct((b, v), jnp.float32),
        scratch_shapes=[
            pltpu.VMEM((b, h), jnp.float32),
            pltpu.VMEM((2, b, block_n), jnp.float32),
            pltpu.VMEM((b, rem_tl), jnp.float32),
            pltpu.SemaphoreType.DMA((2,)),
        ],
    )(e, W1, b1, W2, b2)


def kernel(x, embed, W1, b1, W2, b2):
    b, w = x.shape
    v, d = embed.shape
    h = W1.shape[1]
    idx = x.reshape(-1).astype(jnp.int32).reshape(-1, 128)
    rows = _sc_gather(embed, idx)
    e = rows.reshape(b, w * d)
    return _mlp_manual(
        e,
        W1,
        b1.reshape(1, h),
        W2,
        b2.reshape(1, v),
        block_n=1024,
    )


# speedup vs baseline: 1.0263x; 1.0026x over previous
import functools

import jax
import jax.numpy as jnp
from jax import lax
from jax.experimental import pallas as pl
from jax.experimental.pallas import tpu as pltpu
from jax.experimental.pallas import tpu_sc as plsc


def _sc_gather(table, idx2d):
    nr, il = idx2d.shape
    v, d = table.shape
    info = plsc.get_sparse_core_info()
    nw = info.num_cores * info.num_subcores
    rows_per_w = nr // nw

    mesh = plsc.VectorSubcoreMesh(core_axis_name="c", subcore_axis_name="s")

    @functools.partial(
        pl.kernel,
        mesh=mesh,
        out_type=jax.ShapeDtypeStruct((nr * il, d), table.dtype),
        scratch_types=[
            pltpu.VMEM((rows_per_w, il), jnp.int32),
            pltpu.VMEM((rows_per_w * il, d), table.dtype),
            pltpu.SemaphoreType.DMA,
        ],
    )
    def k(table_hbm, idx_hbm, out_hbm, idx_v, rows_v, sem):
        wid = lax.axis_index("s") * info.num_cores + lax.axis_index("c")
        base = wid * rows_per_w
        pltpu.sync_copy(idx_hbm.at[pl.ds(base, rows_per_w)], idx_v)
        copies = []
        for j in range(rows_per_w):
            copies.append(
                pltpu.async_copy(
                    table_hbm.at[idx_v.at[j]], rows_v.at[pl.ds(j * il, il)], sem
                )
            )
        for c in copies:
            c.wait()
        pltpu.sync_copy(rows_v, out_hbm.at[pl.ds(base * il, rows_per_w * il)])

    return k(table, idx2d)


def _mlp(e, W1, b1, W2, b2, block_m, block_n):
    b, k = e.shape
    h = W1.shape[1]
    v = W2.shape[1]
    nb = b // block_m
    nv = pl.cdiv(v, block_n)

    def body(e_ref, w1_ref, b1_ref, w2_ref, b2_ref, out_ref, h_ref):
        @pl.when(pl.program_id(1) == 0)
        def _():
            h_ref[...] = jnp.tanh(
                jnp.dot(e_ref[...], w1_ref[...], preferred_element_type=jnp.float32)
                + b1_ref[...]
            )

        out_ref[...] = (
            jnp.dot(h_ref[...], w2_ref[...], preferred_element_type=jnp.float32)
            + b2_ref[...]
        )

    return pl.pallas_call(
        body,
        grid=(nb, nv),
        in_specs=[
            pl.BlockSpec((block_m, k), lambda i, j: (i, 0)),
            pl.BlockSpec((k, h), lambda i, j: (0, 0)),
            pl.BlockSpec((1, h), lambda i, j: (0, 0)),
            pl.BlockSpec((h, block_n), lambda i, j: (0, j)),
            pl.BlockSpec((1, block_n), lambda i, j: (0, j)),
        ],
        out_specs=pl.BlockSpec((block_m, block_n), lambda i, j: (i, j)),
        out_shape=jax.ShapeDtypeStruct((b, v), jnp.float32),
        scratch_shapes=[pltpu.VMEM((block_m, h), jnp.float32)],
    )(e, W1, b1, W2, b2)


def kernel(x, embed, W1, b1, W2, b2):
    b, w = x.shape
    v, d = embed.shape
    h = W1.shape[1]
    idx = x.reshape(-1).astype(jnp.int32).reshape(-1, 128)
    rows = _sc_gather(embed, idx)
    e = rows.reshape(b, w * d)
    return _mlp(
        e,
        W1,
        b1.reshape(1, h),
        W2,
        b2.reshape(1, v),
        block_m=b,
        block_n=1024,
    )


# speedup vs baseline: 1.0293x; 1.0029x over previous
import functools

import jax
import jax.numpy as jnp
from jax import lax
from jax.experimental import pallas as pl
from jax.experimental.pallas import tpu as pltpu
from jax.experimental.pallas import tpu_sc as plsc


def _sc_gather(table, idx2d):
    nr, il = idx2d.shape
    v, d = table.shape
    info = plsc.get_sparse_core_info()
    nw = info.num_cores * info.num_subcores
    rows_per_w = nr // nw

    mesh = plsc.VectorSubcoreMesh(core_axis_name="c", subcore_axis_name="s")

    @functools.partial(
        pl.kernel,
        mesh=mesh,
        out_type=jax.ShapeDtypeStruct((nr * il, d), table.dtype),
        scratch_types=[
            pltpu.VMEM((rows_per_w, il), jnp.int32),
            pltpu.VMEM((rows_per_w * il, d), table.dtype),
            pltpu.SemaphoreType.DMA,
            pltpu.SemaphoreType.DMA,
        ],
    )
    def k(table_hbm, idx_hbm, out_hbm, idx_v, rows_v, gsem, ssem):
        wid = lax.axis_index("s") * info.num_cores + lax.axis_index("c")
        base = wid * rows_per_w
        pltpu.sync_copy(idx_hbm.at[pl.ds(base, rows_per_w)], idx_v)
        gathers = []
        for j in range(rows_per_w):
            gathers.append(
                pltpu.async_copy(
                    table_hbm.at[idx_v.at[j]], rows_v.at[pl.ds(j * il, il)], gsem
                )
            )
        scatters = []
        for j in range(rows_per_w):
            gathers[j].wait()
            scatters.append(
                pltpu.async_copy(
                    rows_v.at[pl.ds(j * il, il)],
                    out_hbm.at[pl.ds((base + j) * il, il)],
                    ssem,
                )
            )
        for s in scatters:
            s.wait()

    return k(table, idx2d)


def _mlp(e, W1, b1, W2, b2, block_m, block_n):
    b, k = e.shape
    h = W1.shape[1]
    v = W2.shape[1]
    nb = b // block_m
    nv = pl.cdiv(v, block_n)

    def body(e_ref, w1_ref, b1_ref, w2_ref, b2_ref, out_ref, h_ref):
        @pl.when(pl.program_id(1) == 0)
        def _():
            h_ref[...] = jnp.tanh(
                jnp.dot(e_ref[...], w1_ref[...], preferred_element_type=jnp.float32)
                + b1_ref[...]
            )

        out_ref[...] = (
            jnp.dot(h_ref[...], w2_ref[...], preferred_element_type=jnp.float32)
            + b2_ref[...]
        )

    return pl.pallas_call(
        body,
        grid=(nb, nv),
        in_specs=[
            pl.BlockSpec((block_m, k), lambda i, j: (i, 0)),
            pl.BlockSpec((k, h), lambda i, j: (0, 0)),
            pl.BlockSpec((1, h), lambda i, j: (0, 0)),
            pl.BlockSpec((h, block_n), lambda i, j: (0, j)),
            pl.BlockSpec((1, block_n), lambda i, j: (0, j)),
        ],
        out_specs=pl.BlockSpec((block_m, block_n), lambda i, j: (i, j)),
        out_shape=jax.ShapeDtypeStruct((b, v), jnp.float32),
        scratch_shapes=[pltpu.VMEM((block_m, h), jnp.float32)],
    )(e, W1, b1, W2, b2)


def kernel(x, embed, W1, b1, W2, b2):
    b, w = x.shape
    v, d = embed.shape
    h = W1.shape[1]
    idx = x.reshape(-1).astype(jnp.int32).reshape(-1, 128)
    rows = _sc_gather(embed, idx)
    e = rows.reshape(b, w * d)
    return _mlp(
        e,
        W1,
        b1.reshape(1, h),
        W2,
        b2.reshape(1, v),
        block_m=b,
        block_n=1024,
    )
